# Initial kernel scaffold; baseline (speedup 1.0000x reference)
import jax
import jax.numpy as jnp
from jax.experimental import pallas as pl


def kernel(queries, keys, values, attn_mask):
    raise NotImplementedError("write your pallas kernel here")


# speedup vs baseline: 2.5795x; 2.5795x over previous
import functools
import math

import jax
import jax.numpy as jnp
import numpy as np
from jax import lax
from jax.experimental import pallas as pl
from jax.experimental.pallas import tpu as pltpu
from jax.experimental.pallas import tpu_sc as plsc

FBLK = 256

SC_NC, SC_NS, SC_LANES = 2, 16, 16
SC_NW = SC_NC * SC_NS


@functools.lru_cache(maxsize=None)
def _twiddles(L: int):
    nf_live = L // 2 + 1
    NF = ((nf_live + FBLK - 1) // FBLK) * FBLK
    f = np.arange(NF, dtype=np.int64)
    t = np.arange(L, dtype=np.int64)
    ang = 2.0 * np.pi * ((f[:, None] * t[None, :]) % L).astype(np.float64) / L
    cos = np.cos(ang)
    sin = np.sin(ang)
    live = (f <= L // 2).astype(np.float64)
    cos *= live[:, None]
    sin *= live[:, None]
    w = np.where((f == 0) | (f == L // 2), 1.0, 2.0) / L * live
    icos = (w[:, None] * cos).astype(np.float32)
    isin = (w[:, None] * sin).astype(np.float32)
    fcos = np.ascontiguousarray(cos.T).astype(np.float32)
    fsin = np.ascontiguousarray(sin.T).astype(np.float32)
    return fcos, fsin, icos, isin, NF


def _dot(a, b):
    return jax.lax.dot_general(
        a, b, (((1,), (0,)), ((), ())),
        preferred_element_type=jnp.float32,
        precision=jax.lax.Precision.HIGHEST)


def _corr_body(qfT_ref, kfT_ref, fcos_ref, fsin_ref, icos_ref, isin_ref, c_ref):
    j = pl.program_id(1)
    q = qfT_ref[0]
    k = kfT_ref[0]
    fcos = fcos_ref[...]
    fsin = fsin_ref[...]
    qr = _dot(q, fcos)
    qs = _dot(q, fsin)
    kr = _dot(k, fcos)
    ks = _dot(k, fsin)
    sr = jnp.sum(qr * kr + qs * ks, axis=0, keepdims=True)
    si = jnp.sum(qr * ks - qs * kr, axis=0, keepdims=True)
    contrib = _dot(sr, icos_ref[...]) - _dot(si, isin_ref[...])

    @pl.when(j == 0)
    def _():
        c_ref[0] = contrib

    @pl.when(j > 0)
    def _():
        c_ref[0] = c_ref[0] + contrib


def _corr_mean(qfT, kfT, L):
    B = qfT.shape[0]
    fcos, fsin, icos, isin, NF = _twiddles(L)
    nj = NF // FBLK
    return pl.pallas_call(
        _corr_body,
        grid=(B, nj),
        in_specs=[
            pl.BlockSpec((1, qfT.shape[1], L), lambda b, j: (b, 0, 0)),
            pl.BlockSpec((1, qfT.shape[1], L), lambda b, j: (b, 0, 0)),
            pl.BlockSpec((L, FBLK), lambda b, j: (0, j)),
            pl.BlockSpec((L, FBLK), lambda b, j: (0, j)),
            pl.BlockSpec((FBLK, L), lambda b, j: (j, 0)),
            pl.BlockSpec((FBLK, L), lambda b, j: (j, 0)),
        ],
        out_specs=pl.BlockSpec((1, 1, L), lambda b, j: (b, 0, 0)),
        out_shape=jax.ShapeDtypeStruct((B, 1, L), jnp.float32),
        compiler_params=pltpu.CompilerParams(
            dimension_semantics=("arbitrary", "arbitrary")),
    )(qfT, kfT, jnp.asarray(fcos), jnp.asarray(fsin),
      jnp.asarray(icos), jnp.asarray(isin))


def _topk_body(B, L, HE, TOPK, c_ref, idx_ref, sw_ref):
    c = c_ref[...].reshape(B, L)
    mc = c * (1.0 / HE)
    gm = jnp.sum(mc, axis=0, keepdims=True) * (1.0 / B)
    lane = lax.broadcasted_iota(jnp.int32, (1, L), 1)
    neg = jnp.float32(-jnp.inf)

    cur = gm
    idxs = []
    for _ in range(TOPK):
        m = jnp.max(cur)
        idx = jnp.min(jnp.where(cur == m, lane, L)).astype(jnp.int32)
        idxs.append(idx)
        cur = jnp.where(lane == idx, neg, cur)

    lane128 = lax.broadcasted_iota(jnp.int32, (1, 128), 1)
    for b in range(B):
        mc_b = mc[b:b + 1, :]
        wrow = jnp.zeros((1, 128), jnp.float32)
        for kk in range(TOPK):
            w_bk = jnp.sum(jnp.where(lane == idxs[kk], mc_b, 0.0))
            wrow = jnp.where(lane128 == kk, w_bk, wrow)
        wrow = jnp.where(lane128 < TOPK, wrow, neg)
        mb = jnp.max(wrow)
        e = jnp.exp(wrow - mb)
        s = jnp.sum(e)
        swrow = e / s
        for kk in range(TOPK):
            sw_bk = jnp.sum(jnp.where(lane128 == kk, swrow, 0.0))
            sw_ref[pl.ds(b * TOPK + kk, 1), :] = jnp.full((1, SC_LANES), sw_bk)

    for kk in range(TOPK):
        s_k = idxs[kk]
        rel = jnp.where(lane >= s_k, lane - s_k, lane + (L - s_k))
        for b in range(B):
            idx_ref[pl.ds(b * TOPK + kk, 1), :] = rel + b * L


def _topk_weights(c3, B, L, HE, TOPK):
    body = functools.partial(_topk_body, B, L, HE, TOPK)
    return pl.pallas_call(
        body,
        grid=(1,),
        in_specs=[pl.BlockSpec((B, 1, L), lambda i: (0, 0, 0))],
        out_specs=[
            pl.BlockSpec((B * TOPK, L), lambda i: (0, 0)),
            pl.BlockSpec((B * TOPK, SC_LANES), lambda i: (0, 0)),
        ],
        out_shape=[
            jax.ShapeDtypeStruct((B * TOPK, L), jnp.int32),
            jax.ShapeDtypeStruct((B * TOPK, SC_LANES), jnp.float32),
        ],
    )(c3)


def _sc_gather_agg(vflat, src_idx, sw16, B, L, HE, TOPK):
    ROWS = B * L
    RW = ROWS // SC_NW
    RT = 16
    NSUB = RW // RT
    mesh = plsc.VectorSubcoreMesh(core_axis_name="c", subcore_axis_name="s")
    scratch = ([pltpu.VMEM((RT,), jnp.int32) for _ in range(TOPK)]
               + [pltpu.VMEM((RT, HE), jnp.float32) for _ in range(TOPK)]
               + [pltpu.VMEM((RT, HE), jnp.float32),
                  pltpu.VMEM((B * TOPK, SC_LANES), jnp.float32),
                  pltpu.SemaphoreType.DMA,
                  pltpu.SemaphoreType.DMA,
                  pltpu.SemaphoreType.DMA])

    @functools.partial(
        pl.kernel,
        out_type=jax.ShapeDtypeStruct((ROWS, HE), jnp.float32),
        mesh=mesh,
        scratch_types=scratch)
    def k(v_hbm, idx_hbm, sw_hbm, out_hbm, *sc):
        idxb = sc[0:TOPK]
        rowb = sc[TOPK:2 * TOPK]
        acc = sc[2 * TOPK]
        sw_all = sc[2 * TOPK + 1]
        sem_i, sem_g, sem_o = sc[2 * TOPK + 2], sc[2 * TOPK + 3], sc[2 * TOPK + 4]

        wid = lax.axis_index("c") * SC_NS + lax.axis_index("s")
        base = wid * RW
        b = base // L
        lbase = base - b * L
        pltpu.sync_copy(sw_hbm, sw_all)

        @pl.loop(0, NSUB)
        def _(st):
            loc = lbase + st * RT
            idx_cps = []
            for kk in range(TOPK):
                idx_cps.append(pltpu.async_copy(
                    idx_hbm.at[b * TOPK + kk, pl.ds(loc, RT)], idxb[kk], sem_i))
            gat_cps = []
            for kk in range(TOPK):
                idx_cps[kk].wait()
                gat_cps.append(pltpu.async_copy(
                    v_hbm.at[idxb[kk]], rowb[kk], sem_g))
            for kk in range(TOPK):
                gat_cps[kk].wait()
                wv = sw_all[b * TOPK + kk]
                rb = rowb[kk]
                if kk == 0:
                    @pl.loop(0, RT)
                    def _(r):
                        @pl.loop(0, HE, step=SC_LANES)
                        def _(cc):
                            acc[r, pl.ds(cc, SC_LANES)] = (
                                wv * rb[r, pl.ds(cc, SC_LANES)])
                else:
                    @pl.loop(0, RT)
                    def _(r):
                        @pl.loop(0, HE, step=SC_LANES)
                        def _(cc):
                            plsc.addupdate(
                                acc.at[r, pl.ds(cc, SC_LANES)],
                                wv * rb[r, pl.ds(cc, SC_LANES)])
            pltpu.async_copy(acc, out_hbm.at[pl.ds(base + st * RT, RT)],
                             sem_o).wait()

    return k(vflat, src_idx, sw16)


def kernel(queries, keys, values, attn_mask):
    B, L, H, E = queries.shape
    HE = H * E
    TOPK = max(1, min(int(1 * math.log(L)), L))

    qfT = jnp.transpose(queries.reshape(B, L, HE), (0, 2, 1))
    kfT = jnp.transpose(keys.reshape(B, L, HE), (0, 2, 1))

    c3 = _corr_mean(qfT, kfT, L)
    src_idx, sw16 = _topk_weights(c3, B, L, HE, TOPK)

    vflat = values.reshape(B * L, HE)
    out_flat = _sc_gather_agg(vflat, src_idx, sw16, B, L, HE, TOPK)
    return out_flat.reshape(B, L, H, E)
